# baseline (speedup 1.0000x reference)
import jax
import jax.numpy as jnp
from jax.experimental import pallas as pl
from jax.experimental.pallas import tpu as pltpu

_TF = 256


def _mlp_kernel(x_ref, wg_ref, wu_ref, wd_ref, o_ref, xb_ref):
    j = pl.program_id(0)

    @pl.when(j == 0)
    def _cast_x():
        xb_ref[...] = x_ref[...].astype(jnp.bfloat16)

    x = xb_ref[...]
    g = jnp.dot(x, wg_ref[...].astype(jnp.bfloat16),
                preferred_element_type=jnp.float32)
    u = jnp.dot(x, wu_ref[...].astype(jnp.bfloat16),
                preferred_element_type=jnp.float32)
    a = (g * jax.nn.sigmoid(g)) * u
    p = jnp.dot(a.astype(jnp.bfloat16), wd_ref[...].astype(jnp.bfloat16),
                preferred_element_type=jnp.float32)

    @pl.when(j == 0)
    def _init():
        o_ref[...] = p

    @pl.when(j > 0)
    def _acc():
        o_ref[...] += p


@jax.jit
def kernel(hidden_states, router_w, w_gate_proj, w_up_proj, w_down_proj):
    b, s, d = hidden_states.shape
    n = b * s
    ff = w_gate_proj.shape[1]
    x = hidden_states.reshape(n, d)
    out = pl.pallas_call(
        _mlp_kernel,
        grid=(ff // _TF,),
        in_specs=[
            pl.BlockSpec((n, d), lambda j: (0, 0)),
            pl.BlockSpec((d, _TF), lambda j: (0, j)),
            pl.BlockSpec((d, _TF), lambda j: (0, j)),
            pl.BlockSpec((_TF, d), lambda j: (j, 0)),
        ],
        out_specs=pl.BlockSpec((n, d), lambda j: (0, 0)),
        out_shape=jax.ShapeDtypeStruct((n, d), jnp.float32),
        scratch_shapes=[pltpu.VMEM((n, d), jnp.bfloat16)],
    )(x, w_gate_proj, w_up_proj, w_down_proj)
    return out.reshape(b, s, d)


# speedup vs baseline: 1.1366x; 1.1366x over previous
import jax
import jax.numpy as jnp
from jax.experimental import pallas as pl
from jax.experimental.pallas import tpu as pltpu

_TN = 512
_TF = 512


def _mlp_kernel(x_hbm, wg_hbm, wu_hbm, wd_hbm, o_hbm,
                xs, xb, wgs, wus, wds, wdb, ab,
                sx, swg, swu, swd, so):
    n, d = xs.shape
    ff = wgs.shape[1]
    ni, nf = n // _TN, ff // _TF

    cx = [pltpu.make_async_copy(x_hbm.at[pl.ds(i * _TN, _TN), :],
                                xs.at[pl.ds(i * _TN, _TN), :], sx.at[i])
          for i in range(ni)]
    cwg = [pltpu.make_async_copy(wg_hbm.at[:, pl.ds(f * _TF, _TF)],
                                 wgs.at[:, pl.ds(f * _TF, _TF)], swg.at[f])
           for f in range(nf)]
    cwu = [pltpu.make_async_copy(wu_hbm.at[:, pl.ds(f * _TF, _TF)],
                                 wus.at[:, pl.ds(f * _TF, _TF)], swu.at[f])
           for f in range(nf)]
    cwd = [pltpu.make_async_copy(wd_hbm.at[pl.ds(f * _TF, _TF), :],
                                 wds.at[pl.ds(f * _TF, _TF), :], swd.at[f])
           for f in range(nf)]

    cx[0].start()
    cwg[0].start()
    cwu[0].start()
    for i in range(1, ni):
        cx[i].start()
    for f in range(1, nf):
        cwg[f].start()
        cwu[f].start()
    for f in range(nf):
        cwd[f].start()

    for f in range(nf):
        cwg[f].wait()
        cwu[f].wait()
        wgf = wgs[:, f * _TF:(f + 1) * _TF].astype(jnp.bfloat16)
        wuf = wus[:, f * _TF:(f + 1) * _TF].astype(jnp.bfloat16)
        for i in range(ni):
            if f == 0:
                cx[i].wait()
                xb[pl.ds(i * _TN, _TN), :] = (
                    xs[pl.ds(i * _TN, _TN), :].astype(jnp.bfloat16))
            xi = xb[pl.ds(i * _TN, _TN), :]
            g = jnp.dot(xi, wgf, preferred_element_type=jnp.float32)
            u = jnp.dot(xi, wuf, preferred_element_type=jnp.float32)
            a = (g * jax.nn.sigmoid(g)) * u
            ab[pl.ds(i * _TN, _TN), pl.ds(f * _TF, _TF)] = a.astype(jnp.bfloat16)

    co = [pltpu.make_async_copy(xs.at[pl.ds(i * _TN, _TN), :],
                                o_hbm.at[pl.ds(i * _TN, _TN), :], so.at[i])
          for i in range(ni)]
    for i in range(ni):
        acc = None
        for f in range(nf):
            if i == 0:
                cwd[f].wait()
                wdb[pl.ds(f * _TF, _TF), :] = (
                    wds[pl.ds(f * _TF, _TF), :].astype(jnp.bfloat16))
            p = jnp.dot(ab[pl.ds(i * _TN, _TN), pl.ds(f * _TF, _TF)],
                        wdb[pl.ds(f * _TF, _TF), :],
                        preferred_element_type=jnp.float32)
            acc = p if acc is None else acc + p
        xs[pl.ds(i * _TN, _TN), :] = acc
        co[i].start()
    for i in range(ni):
        co[i].wait()


@jax.jit
def kernel(hidden_states, router_w, w_gate_proj, w_up_proj, w_down_proj):
    b, s, d = hidden_states.shape
    n = b * s
    ff = w_gate_proj.shape[1]
    x = hidden_states.reshape(n, d)
    hbm = pl.BlockSpec(memory_space=pltpu.MemorySpace.HBM)
    out = pl.pallas_call(
        _mlp_kernel,
        in_specs=[hbm, hbm, hbm, hbm],
        out_specs=hbm,
        out_shape=jax.ShapeDtypeStruct((n, d), jnp.float32),
        scratch_shapes=[
            pltpu.VMEM((n, d), jnp.float32),
            pltpu.VMEM((n, d), jnp.bfloat16),
            pltpu.VMEM((d, ff), jnp.float32),
            pltpu.VMEM((d, ff), jnp.float32),
            pltpu.VMEM((ff, d), jnp.float32),
            pltpu.VMEM((ff, d), jnp.bfloat16),
            pltpu.VMEM((n, ff), jnp.bfloat16),
            pltpu.SemaphoreType.DMA((n // _TN,)),
            pltpu.SemaphoreType.DMA((ff // _TF,)),
            pltpu.SemaphoreType.DMA((ff // _TF,)),
            pltpu.SemaphoreType.DMA((ff // _TF,)),
            pltpu.SemaphoreType.DMA((n // _TN,)),
        ],
    )(x, w_gate_proj, w_up_proj, w_down_proj)
    return out.reshape(b, s, d)


# speedup vs baseline: 1.2876x; 1.1329x over previous
import jax
import jax.numpy as jnp
from jax.experimental import pallas as pl
from jax.experimental.pallas import tpu as pltpu

_TN = 512
_TF = 512


def _mlp_kernel(x_hbm, wg_hbm, wu_hbm, wd_hbm, o_hbm,
                xs, xb, wgs, wus, wds, wdb, ab,
                sx, swg, swu, swd, so):
    n, d = xs.shape
    ff = wgs.shape[1]
    ni, nf = n // _TN, ff // _TF

    cx = [pltpu.make_async_copy(x_hbm.at[pl.ds(i * _TN, _TN), :],
                                xs.at[pl.ds(i * _TN, _TN), :], sx.at[i])
          for i in range(ni)]
    cwg = [pltpu.make_async_copy(wg_hbm.at[:, pl.ds(f * _TF, _TF)],
                                 wgs.at[:, pl.ds(f * _TF, _TF)], swg.at[f])
           for f in range(nf)]
    cwu = [pltpu.make_async_copy(wu_hbm.at[:, pl.ds(f * _TF, _TF)],
                                 wus.at[:, pl.ds(f * _TF, _TF)], swu.at[f])
           for f in range(nf)]
    cwd = pltpu.make_async_copy(wd_hbm, wds, swd)

    cx[0].start()
    cwg[0].start()
    cwu[0].start()
    for i in range(1, ni):
        cx[i].start()
    for f in range(1, nf):
        cwg[f].start()
        cwu[f].start()
    cwd.start()

    for f in range(nf):
        cwg[f].wait()
        cwu[f].wait()
        wgf = wgs[:, f * _TF:(f + 1) * _TF].astype(jnp.bfloat16)
        wuf = wus[:, f * _TF:(f + 1) * _TF].astype(jnp.bfloat16)
        for i in range(ni):
            if f == 0:
                cx[i].wait()
                xb[pl.ds(i * _TN, _TN), :] = (
                    xs[pl.ds(i * _TN, _TN), :].astype(jnp.bfloat16))
            xi = xb[pl.ds(i * _TN, _TN), :]
            g = jnp.dot(xi, wgf, preferred_element_type=jnp.float32)
            u = jnp.dot(xi, wuf, preferred_element_type=jnp.float32)
            a = (g * jax.nn.sigmoid(g)) * u
            ab[pl.ds(i * _TN, _TN), pl.ds(f * _TF, _TF)] = a.astype(jnp.bfloat16)

    cwd.wait()
    wdb[...] = wds[...].astype(jnp.bfloat16)
    co = [pltpu.make_async_copy(xs.at[pl.ds(i * _TN, _TN), :],
                                o_hbm.at[pl.ds(i * _TN, _TN), :], so.at[i])
          for i in range(ni)]
    for i in range(ni):
        xs[pl.ds(i * _TN, _TN), :] = jnp.dot(
            ab[pl.ds(i * _TN, _TN), :], wdb[...],
            preferred_element_type=jnp.float32)
        co[i].start()
    for i in range(ni):
        co[i].wait()


@jax.jit
def kernel(hidden_states, router_w, w_gate_proj, w_up_proj, w_down_proj):
    b, s, d = hidden_states.shape
    n = b * s
    ff = w_gate_proj.shape[1]
    x = hidden_states.reshape(n, d)
    hbm = pl.BlockSpec(memory_space=pltpu.MemorySpace.HBM)
    out = pl.pallas_call(
        _mlp_kernel,
        in_specs=[hbm, hbm, hbm, hbm],
        out_specs=hbm,
        out_shape=jax.ShapeDtypeStruct((n, d), jnp.float32),
        scratch_shapes=[
            pltpu.VMEM((n, d), jnp.float32),
            pltpu.VMEM((n, d), jnp.bfloat16),
            pltpu.VMEM((d, ff), jnp.float32),
            pltpu.VMEM((d, ff), jnp.float32),
            pltpu.VMEM((ff, d), jnp.float32),
            pltpu.VMEM((ff, d), jnp.bfloat16),
            pltpu.VMEM((n, ff), jnp.bfloat16),
            pltpu.SemaphoreType.DMA((n // _TN,)),
            pltpu.SemaphoreType.DMA((ff // _TF,)),
            pltpu.SemaphoreType.DMA((ff // _TF,)),
            pltpu.SemaphoreType.DMA,
            pltpu.SemaphoreType.DMA((n // _TN,)),
        ],
    )(x, w_gate_proj, w_up_proj, w_down_proj)
    return out.reshape(b, s, d)
